# baseline (speedup 1.0000x reference)
import jax
import jax.numpy as jnp
from jax import lax
from jax.experimental import pallas as pl
from jax.experimental.pallas import tpu as pltpu
from jax.experimental.pallas import tpu_sc as plsc

NUM_CORES = 2
NUM_SUBCORES = 16
NUM_WORKERS = NUM_CORES * NUM_SUBCORES
LANES = 16
NBUF = 4


def _build(batch, seq, feat):
    bpw = batch // NUM_WORKERS
    ftiles = feat // 8
    dims = (seq, ftiles, batch // bpw, 8, bpw)
    assert bpw == 128 and seq % NBUF == 0 and seq >= 3 * NBUF

    def body(idx_hbm, table_hbm, out_hbm, idx_v, idx_t, gbuf, tbuf, gsems, ssems):
        wid = lax.axis_index("s") * NUM_CORES + lax.axis_index("c")
        iota = lax.iota(jnp.int32, LANES)

        pltpu.sync_copy(idx_hbm.at[pl.ds(wid * bpw, bpw)], idx_v)

        @pl.loop(0, seq)
        def _(s):
            for b0 in range(0, bpw, LANES):
                v = plsc.load_gather(idx_v, [b0 + iota, jnp.full((LANES,), s, jnp.int32)])
                idx_t[s, pl.ds(b0, LANES)] = v

        def fire_gather(s, k):
            pltpu.async_copy(table_hbm.at[idx_t.at[s]], gbuf.at[k], gsems.at[k])

        def wait_gather(k):
            pltpu.make_async_copy(
                table_hbm.at[pl.ds(0, bpw)], gbuf.at[k], gsems.at[k]
            ).wait()

        def transpose(k):
            for tf in range(ftiles):
                @pl.loop(0, 8)
                def _(fs):
                    fvec = jnp.full((LANES,), tf * 8 + fs, jnp.int32)
                    for b0 in range(0, bpw, LANES):
                        v = plsc.load_gather(gbuf.at[k], [b0 + iota, fvec])
                        tbuf[k, tf, fs, pl.ds(b0, LANES)] = v

        def fire_store(s, k):
            for tf in range(ftiles):
                pltpu.async_copy(
                    tbuf.at[k, tf], out_hbm.at[s, tf, wid], ssems.at[k]
                )

        def wait_store(k):
            pltpu.make_async_copy(
                tbuf.at[k], out_hbm.at[0, :, 0], ssems.at[k]
            ).wait()

        for s in range(NBUF):
            fire_gather(s, s)
        for s in range(NBUF):
            wait_gather(s)
            transpose(s)
            fire_store(s, s)
            fire_gather(s + NBUF, s)

        @pl.loop(NBUF, seq - NBUF, step=NBUF)
        def _(t):
            for k in range(NBUF):
                s = t + k
                wait_gather(k)
                wait_store(k)
                transpose(k)
                fire_store(s, k)
                fire_gather(s + NBUF, k)

        for i in range(NBUF):
            s = seq - NBUF + i
            wait_gather(i)
            wait_store(i)
            transpose(i)
            fire_store(s, i)
        for k in range(NBUF):
            wait_store(k)

    return pl.kernel(
        body,
        out_type=jax.ShapeDtypeStruct(dims, jnp.float32),
        mesh=plsc.VectorSubcoreMesh(core_axis_name="c", subcore_axis_name="s"),
        scratch_types=[
            pltpu.VMEM((bpw, seq), jnp.int32),
            pltpu.VMEM((seq, bpw), jnp.int32),
            pltpu.VMEM((NBUF, bpw, feat), jnp.float32),
            pltpu.VMEM((NBUF, ftiles, 8, bpw), jnp.float32),
            pltpu.SemaphoreType.DMA((NBUF,)),
            pltpu.SemaphoreType.DMA((NBUF,)),
        ],
        compiler_params=pltpu.CompilerParams(use_tc_tiling_on_sc=False, needs_layout_passes=False),
    )


def kernel(inputs, embedding):
    batch, seq = inputs.shape
    _, feat = embedding.shape
    out5 = _build(batch, seq, feat)(inputs, embedding)
    return out5.transpose(2, 4, 0, 1, 3).reshape(batch, seq, feat)
